# baseline (speedup 1.0000x reference)
import functools
import math

import jax
import jax.numpy as jnp
from jax import lax
from jax.experimental import pallas as pl
from jax.experimental.pallas import tpu as pltpu
from jax.experimental.pallas import tpu_sc as plsc


def _gelu_exact(v):
    return 0.5 * v * (1.0 + jax.lax.erf(v / jnp.float32(math.sqrt(2.0))))


def _router_body(x_ref, q1w_ref, q1b_ref, lnw_ref, lnb_ref, q2w_ref, q2b_ref,
                 nk_ref, mask_ref, *, k_sel):
    gmax = jnp.max(x_ref[0], axis=0, keepdims=True)
    h = jax.lax.dot_general(gmax, q1w_ref[...], (((1,), (1,)), ((), ())),
                            preferred_element_type=jnp.float32) + q1b_ref[...]
    h = _gelu_exact(h)
    mu = jnp.mean(h, axis=-1, keepdims=True)
    var = jnp.mean((h - mu) ** 2, axis=-1, keepdims=True)
    h = (h - mu) / jnp.sqrt(var + 1e-5) * lnw_ref[...] + lnb_ref[...]
    q = jax.lax.dot_general(h, q2w_ref[...], (((1,), (1,)), ((), ())),
                            preferred_element_type=jnp.float32) + q2b_ref[...]
    d_routing = q.shape[-1]
    logits = jax.lax.dot_general(q, nk_ref[...], (((1,), (1,)), ((), ())),
                                 preferred_element_type=jnp.float32)
    logits = logits / jnp.float32(math.sqrt(d_routing))

    n_in = logits.shape[-1]
    u = jax.lax.bitcast_convert_type(logits, jnp.uint32)
    msb = jnp.uint32(0x80000000)
    ukey = jnp.where(u >= msb, ~u, u | msb)

    def tbody(i, t):
        cand = t | (jnp.uint32(1) << jnp.uint32(31 - i))
        cnt = jnp.sum((ukey >= cand).astype(jnp.int32))
        return jnp.where(cnt >= k_sel, cand, t)

    t = jax.lax.fori_loop(0, 32, tbody, jnp.uint32(0))

    c_gt = jnp.sum((ukey > t).astype(jnp.int32))
    need = k_sel - c_gt
    idx = jax.lax.broadcasted_iota(jnp.int32, (1, n_in), 1)
    eq = ukey == t

    nbits = max(1, (n_in - 1).bit_length())

    def jbody(i, lh):
        lo, hi = lh
        mid = (lo + hi) // 2
        g = jnp.sum((eq & (idx <= mid)).astype(jnp.int32))
        pred = g >= need
        return jnp.where(pred, lo, mid + 1), jnp.where(pred, mid, hi)

    lo, _ = jax.lax.fori_loop(0, nbits, jbody,
                              (jnp.int32(0), jnp.int32(n_in - 1)))
    sel = (ukey > t) | (eq & (idx <= lo) & (need > 0))
    mask_ref[0] = sel.astype(jnp.float32)


def _transpose_body(w_ref, wt_ref):
    wt_ref[...] = w_ref[...].T


def _sc_gather_one(mask_b, ip, pwt, k_sel):
    _, n_in = mask_b.shape
    D = ip.shape[1]
    n_pr = pwt.shape[1]
    info = plsc.get_sparse_core_info()
    nw = info.num_cores * info.num_subcores
    rows_w = k_sel // nw
    ch = min(32, rows_w)
    nc = info.num_cores
    mesh = plsc.VectorSubcoreMesh(core_axis_name="c", subcore_axis_name="s")

    @functools.partial(
        pl.kernel,
        out_type=(jax.ShapeDtypeStruct((k_sel, D), jnp.float32),
                  jax.ShapeDtypeStruct((k_sel, n_pr), jnp.float32)),
        mesh=mesh,
        scratch_types=[
            pltpu.VMEM((n_in,), jnp.float32),
            pltpu.VMEM((k_sel,), jnp.int32),
            pltpu.VMEM((ch, D), jnp.float32),
            pltpu.VMEM((ch, n_pr), jnp.float32),
            pltpu.SemaphoreType.DMA,
        ],
        compiler_params=pltpu.CompilerParams(needs_layout_passes=False),
    )
    def sc_kernel(mask_hbm, ip_hbm, pwt_hbm, ipsel_hbm, pwtsel_hbm,
                  mask_v, idx_v, ip_buf, pwt_buf, sem):
        g = lax.axis_index("s") * nc + lax.axis_index("c")
        pltpu.sync_copy(mask_hbm.at[0], mask_v)

        def cbody(i, off):
            half = jnp.full((16,), 0.5, jnp.float32)
            m = mask_v[pl.ds(i * 16, 16)] > half
            mi = jnp.where(m, jnp.full((16,), 1, jnp.int32),
                           jnp.full((16,), 0, jnp.int32))
            lanes = lax.iota(jnp.int32, 16) + jnp.broadcast_to(i * 16, (16,))
            pos = jnp.broadcast_to(off, (16,)) + plsc.cumsum(mi) - mi
            plsc.store_scatter(idx_v, [pos], lanes, mask=m)
            return off + jnp.sum(mi)

        lax.fori_loop(0, n_in // 16, cbody, jnp.int32(0))

        base = g * rows_w
        for c in range(rows_w // ch):
            r0 = base + c * ch
            idx_slice = idx_v.at[pl.ds(r0, ch)]
            pltpu.async_copy(ip_hbm.at[idx_slice], ip_buf, sem).wait()
            pltpu.sync_copy(ip_buf, ipsel_hbm.at[pl.ds(r0, ch)])
            pltpu.async_copy(pwt_hbm.at[idx_slice], pwt_buf, sem).wait()
            pltpu.sync_copy(pwt_buf, pwtsel_hbm.at[pl.ds(r0, ch)])

    return sc_kernel(mask_b, ip, pwt)


def _ffn_body(x_ref, ipsel_ref, pwtsel_ref, po_ref, out_ref, acc_ref):
    k = pl.program_id(1)
    nk = pl.num_programs(1)
    acts = jax.lax.dot_general(x_ref[...], ipsel_ref[...],
                               (((1,), (1,)), ((), ())),
                               preferred_element_type=jnp.float32)
    acts = _gelu_exact(acts)
    contrib = jax.lax.dot_general(acts, pwtsel_ref[...],
                                  (((1,), (0,)), ((), ())),
                                  preferred_element_type=jnp.float32)

    @pl.when(k == 0)
    def _():
        acc_ref[...] = contrib

    @pl.when(k > 0)
    def _():
        acc_ref[...] += contrib

    @pl.when(k == nk - 1)
    def _():
        pacts = _gelu_exact(acc_ref[...])
        out_ref[...] = jax.lax.dot_general(pacts, po_ref[...],
                                           (((1,), (0,)), ((), ())),
                                           preferred_element_type=jnp.float32)


def kernel(x, k_input, k_process, q1_w, q1_b, ln_w, ln_b, q2_w, q2_b,
           neuron_keys, input_patterns, process_weights, process_outputs):
    B, S, D = x.shape
    N_IN, D_R = neuron_keys.shape
    N_PR = process_weights.shape[0]
    K_SEL = N_PR

    mask = pl.pallas_call(
        functools.partial(_router_body, k_sel=K_SEL),
        grid=(B,),
        in_specs=[
            pl.BlockSpec((1, S, D), lambda b: (b, 0, 0)),
            pl.BlockSpec(q1_w.shape, lambda b: (0, 0)),
            pl.BlockSpec((1, q1_b.shape[0]), lambda b: (0, 0)),
            pl.BlockSpec((1, ln_w.shape[0]), lambda b: (0, 0)),
            pl.BlockSpec((1, ln_b.shape[0]), lambda b: (0, 0)),
            pl.BlockSpec(q2_w.shape, lambda b: (0, 0)),
            pl.BlockSpec((1, q2_b.shape[0]), lambda b: (0, 0)),
            pl.BlockSpec(neuron_keys.shape, lambda b: (0, 0)),
        ],
        out_specs=pl.BlockSpec((1, 1, N_IN), lambda b: (b, 0, 0)),
        out_shape=jax.ShapeDtypeStruct((B, 1, N_IN), jnp.float32),
    )(x, q1_w, q1_b.reshape(1, -1), ln_w.reshape(1, -1), ln_b.reshape(1, -1),
      q2_w, q2_b.reshape(1, -1), neuron_keys)

    TB = 512
    pwt = pl.pallas_call(
        _transpose_body,
        grid=(N_IN // TB, N_PR // TB),
        in_specs=[pl.BlockSpec((TB, TB), lambda i, j: (j, i))],
        out_specs=pl.BlockSpec((TB, TB), lambda i, j: (i, j)),
        out_shape=jax.ShapeDtypeStruct((N_IN, N_PR), jnp.float32),
    )(process_weights)

    mask2d = mask.reshape(B, N_IN)

    TS = min(512, S)
    KT = min(1024, K_SEL)
    outs = []
    for b in range(B):
        ip_sel, pwt_sel = _sc_gather_one(mask2d[b:b + 1], input_patterns,
                                         pwt, K_SEL)
        out_b = pl.pallas_call(
            _ffn_body,
            grid=(S // TS, K_SEL // KT),
            in_specs=[
                pl.BlockSpec((TS, D), lambda s, k: (s, 0)),
                pl.BlockSpec((KT, D), lambda s, k: (k, 0)),
                pl.BlockSpec((KT, N_PR), lambda s, k: (k, 0)),
                pl.BlockSpec((N_PR, D), lambda s, k: (0, 0)),
            ],
            out_specs=pl.BlockSpec((TS, D), lambda s, k: (s, 0)),
            out_shape=jax.ShapeDtypeStruct((S, D), jnp.float32),
            scratch_shapes=[pltpu.VMEM((TS, N_PR), jnp.float32)],
            compiler_params=pltpu.CompilerParams(
                dimension_semantics=("arbitrary", "arbitrary"),
            ),
        )(x[b], ip_sel, pwt_sel, process_outputs)
        outs.append(out_b)
    return jnp.stack(outs, axis=0)


# speedup vs baseline: 1.1372x; 1.1372x over previous
import functools
import math

import jax
import jax.numpy as jnp
from jax import lax
from jax.experimental import pallas as pl
from jax.experimental.pallas import tpu as pltpu
from jax.experimental.pallas import tpu_sc as plsc


def _gelu_exact(v):
    return 0.5 * v * (1.0 + jax.lax.erf(v / jnp.float32(math.sqrt(2.0))))


def _router_body(x_ref, q1w_ref, q1b_ref, lnw_ref, lnb_ref, q2w_ref, q2b_ref,
                 nk_ref, mask_ref, *, k_sel):
    gmax = jnp.max(x_ref[0], axis=0, keepdims=True)
    h = jax.lax.dot_general(gmax, q1w_ref[...], (((1,), (1,)), ((), ())),
                            preferred_element_type=jnp.float32) + q1b_ref[...]
    h = _gelu_exact(h)
    mu = jnp.mean(h, axis=-1, keepdims=True)
    var = jnp.mean((h - mu) ** 2, axis=-1, keepdims=True)
    h = (h - mu) / jnp.sqrt(var + 1e-5) * lnw_ref[...] + lnb_ref[...]
    q = jax.lax.dot_general(h, q2w_ref[...], (((1,), (1,)), ((), ())),
                            preferred_element_type=jnp.float32) + q2b_ref[...]
    d_routing = q.shape[-1]
    logits = jax.lax.dot_general(q, nk_ref[...], (((1,), (1,)), ((), ())),
                                 preferred_element_type=jnp.float32)
    logits = logits / jnp.float32(math.sqrt(d_routing))

    n_in = logits.shape[-1]
    u = jax.lax.bitcast_convert_type(logits, jnp.uint32)
    msb = jnp.uint32(0x80000000)
    ukey = jnp.where(u >= msb, ~u, u | msb)

    def tbody(i, t):
        cand = t | (jnp.uint32(1) << jnp.uint32(31 - i))
        cnt = jnp.sum((ukey >= cand).astype(jnp.int32))
        return jnp.where(cnt >= k_sel, cand, t)

    t = jax.lax.fori_loop(0, 32, tbody, jnp.uint32(0))

    c_gt = jnp.sum((ukey > t).astype(jnp.int32))
    need = k_sel - c_gt
    idx = jax.lax.broadcasted_iota(jnp.int32, (1, n_in), 1)
    eq = ukey == t

    nbits = max(1, (n_in - 1).bit_length())

    def jbody(i, lh):
        lo, hi = lh
        mid = (lo + hi) // 2
        g = jnp.sum((eq & (idx <= mid)).astype(jnp.int32))
        pred = g >= need
        return jnp.where(pred, lo, mid + 1), jnp.where(pred, mid, hi)

    lo, _ = jax.lax.fori_loop(0, nbits, jbody,
                              (jnp.int32(0), jnp.int32(n_in - 1)))
    sel = (ukey > t) | (eq & (idx <= lo) & (need > 0))
    mask_ref[0] = sel.astype(jnp.float32)


def _transpose_body(w_ref, wt_ref):
    wt_ref[...] = w_ref[...].T


def _sc_gather(mask2d, ip, pwt, k_sel):
    B, n_in = mask2d.shape
    D = ip.shape[1]
    n_pr = pwt.shape[1]
    info = plsc.get_sparse_core_info()
    nw = info.num_cores * info.num_subcores
    gpb = nw // B
    rows_w = k_sel // gpb
    ch = min(16, rows_w)
    nch = rows_w // ch
    nc = info.num_cores
    mesh = plsc.VectorSubcoreMesh(core_axis_name="c", subcore_axis_name="s")

    @functools.partial(
        pl.kernel,
        out_type=(jax.ShapeDtypeStruct((B, k_sel, D), jnp.float32),
                  jax.ShapeDtypeStruct((B, k_sel, n_pr), jnp.float32)),
        mesh=mesh,
        scratch_types=[
            pltpu.VMEM((n_in,), jnp.float32),
            pltpu.VMEM((k_sel,), jnp.int32),
            pltpu.VMEM((2, ch, D), jnp.float32),
            pltpu.VMEM((2, ch, n_pr), jnp.float32),
            pltpu.SemaphoreType.DMA,
            pltpu.SemaphoreType.DMA,
            pltpu.SemaphoreType.DMA,
            pltpu.SemaphoreType.DMA,
        ],
        compiler_params=pltpu.CompilerParams(needs_layout_passes=False),
    )
    def sc_kernel(mask_hbm, ip_hbm, pwt_hbm, ipsel_hbm, pwtsel_hbm,
                  mask_v, idx_v, ip_buf, pwt_buf,
                  sem_gi, sem_gp, sem_wi, sem_wp):
        wid = lax.axis_index("s") * nc + lax.axis_index("c")
        b = wid // gpb
        g = wid % gpb
        pltpu.sync_copy(mask_hbm.at[b], mask_v)

        def cbody(i, off):
            half = jnp.full((16,), 0.5, jnp.float32)
            m = mask_v[pl.ds(i * 16, 16)] > half
            mi = jnp.where(m, jnp.full((16,), 1, jnp.int32),
                           jnp.full((16,), 0, jnp.int32))
            lanes = lax.iota(jnp.int32, 16) + jnp.broadcast_to(i * 16, (16,))
            pos = jnp.broadcast_to(off, (16,)) + plsc.cumsum(mi) - mi
            plsc.store_scatter(idx_v, [pos], lanes, mask=m)
            return off + jnp.sum(mi)

        lax.fori_loop(0, n_in // 16, cbody, jnp.int32(0))

        base = g * rows_w

        def gathers(c):
            r0 = base + c * ch
            sl = idx_v.at[pl.ds(r0, ch)]
            hi = pltpu.async_copy(ip_hbm.at[sl], ip_buf.at[c % 2], sem_gi)
            hp = pltpu.async_copy(pwt_hbm.at[sl], pwt_buf.at[c % 2], sem_gp)
            return hi, hp

        def writes(c):
            r0 = base + c * ch
            hi = pltpu.async_copy(ip_buf.at[c % 2],
                                  ipsel_hbm.at[b, pl.ds(r0, ch)], sem_wi)
            hp = pltpu.async_copy(pwt_buf.at[c % 2],
                                  pwtsel_hbm.at[b, pl.ds(r0, ch)], sem_wp)
            return hi, hp

        g_h = [None] * nch
        w_h = [None] * nch
        g_h[0] = gathers(0)
        for c in range(nch):
            g_h[c][0].wait()
            g_h[c][1].wait()
            w_h[c] = writes(c)
            if c + 1 < nch:
                if c >= 1:
                    w_h[c - 1][0].wait()
                    w_h[c - 1][1].wait()
                g_h[c + 1] = gathers(c + 1)
        if nch >= 2:
            w_h[nch - 2][0].wait()
            w_h[nch - 2][1].wait()
        w_h[nch - 1][0].wait()
        w_h[nch - 1][1].wait()

    return sc_kernel(mask2d, ip, pwt)


def _ffn_body(x_ref, ipsel_ref, pwtsel_ref, po_ref, out_ref, acc_ref):
    k = pl.program_id(2)
    nk = pl.num_programs(2)
    acts = jax.lax.dot_general(x_ref[0], ipsel_ref[0],
                               (((1,), (1,)), ((), ())),
                               preferred_element_type=jnp.float32)
    acts = _gelu_exact(acts)
    contrib = jax.lax.dot_general(acts, pwtsel_ref[0],
                                  (((1,), (0,)), ((), ())),
                                  preferred_element_type=jnp.float32)

    @pl.when(k == 0)
    def _():
        acc_ref[...] = contrib

    @pl.when(k > 0)
    def _():
        acc_ref[...] += contrib

    @pl.when(k == nk - 1)
    def _():
        pacts = _gelu_exact(acc_ref[...])
        out_ref[0] = jax.lax.dot_general(pacts, po_ref[...],
                                           (((1,), (0,)), ((), ())),
                                           preferred_element_type=jnp.float32)


def kernel(x, k_input, k_process, q1_w, q1_b, ln_w, ln_b, q2_w, q2_b,
           neuron_keys, input_patterns, process_weights, process_outputs):
    B, S, D = x.shape
    N_IN, D_R = neuron_keys.shape
    N_PR = process_weights.shape[0]
    K_SEL = N_PR

    mask = pl.pallas_call(
        functools.partial(_router_body, k_sel=K_SEL),
        grid=(B,),
        in_specs=[
            pl.BlockSpec((1, S, D), lambda b: (b, 0, 0)),
            pl.BlockSpec(q1_w.shape, lambda b: (0, 0)),
            pl.BlockSpec((1, q1_b.shape[0]), lambda b: (0, 0)),
            pl.BlockSpec((1, ln_w.shape[0]), lambda b: (0, 0)),
            pl.BlockSpec((1, ln_b.shape[0]), lambda b: (0, 0)),
            pl.BlockSpec(q2_w.shape, lambda b: (0, 0)),
            pl.BlockSpec((1, q2_b.shape[0]), lambda b: (0, 0)),
            pl.BlockSpec(neuron_keys.shape, lambda b: (0, 0)),
        ],
        out_specs=pl.BlockSpec((1, 1, N_IN), lambda b: (b, 0, 0)),
        out_shape=jax.ShapeDtypeStruct((B, 1, N_IN), jnp.float32),
    )(x, q1_w, q1_b.reshape(1, -1), ln_w.reshape(1, -1), ln_b.reshape(1, -1),
      q2_w, q2_b.reshape(1, -1), neuron_keys)

    TB = 512
    pwt = pl.pallas_call(
        _transpose_body,
        grid=(N_IN // TB, N_PR // TB),
        in_specs=[pl.BlockSpec((TB, TB), lambda i, j: (j, i))],
        out_specs=pl.BlockSpec((TB, TB), lambda i, j: (i, j)),
        out_shape=jax.ShapeDtypeStruct((N_IN, N_PR), jnp.float32),
    )(process_weights)

    ip_sel, pwt_sel = _sc_gather(mask.reshape(B, N_IN), input_patterns, pwt,
                                 K_SEL)

    TS = min(512, S)
    KT = min(1024, K_SEL)
    out = pl.pallas_call(
        _ffn_body,
        grid=(B, S // TS, K_SEL // KT),
        in_specs=[
            pl.BlockSpec((1, TS, D), lambda b, s, k: (b, s, 0)),
            pl.BlockSpec((1, KT, D), lambda b, s, k: (b, k, 0)),
            pl.BlockSpec((1, KT, N_PR), lambda b, s, k: (b, k, 0)),
            pl.BlockSpec((N_PR, D), lambda b, s, k: (0, 0)),
        ],
        out_specs=pl.BlockSpec((1, TS, D), lambda b, s, k: (b, s, 0)),
        out_shape=jax.ShapeDtypeStruct((B, S, D), jnp.float32),
        scratch_shapes=[pltpu.VMEM((TS, N_PR), jnp.float32)],
        compiler_params=pltpu.CompilerParams(
            dimension_semantics=("parallel", "arbitrary", "arbitrary"),
        ),
    )(x, ip_sel, pwt_sel, process_outputs)
    return out
